# initial kernel scaffold (unmeasured)
import functools

import numpy as np
import jax
import jax.numpy as jnp
from jax import lax
from jax.experimental import pallas as pl
from jax.experimental.pallas import tpu as pltpu

N_DEV = 4
B, SQ, DM = 4, 1024, 1024
HL, DH = 8, 128
SCALE = 0.08838834764831843


def _rope_tables():
    inv = 1.0 / (10000.0 ** (np.arange(0, DH, 2) / DH))
    pos = np.arange(SQ)[:, None] * inv[None, :]
    cos = np.repeat(np.cos(pos), 2, axis=-1).astype(np.float32)
    sin = np.repeat(np.sin(pos), 2, axis=-1).astype(np.float32)
    return np.tile(cos, (1, HL)), np.tile(sin, (1, HL))


def kernel(x, Wq, Wk, Wv, Wo):
    x2 = x.reshape(SQ, DM)
    cos_np, sin_np = _rope_tables()
    cos = jnp.asarray(cos_np)
    sin = jnp.asarray(sin_np)

    def body(x_ref, wq_ref, wk_ref, wv_ref, wo_ref, cos_ref, sin_ref,
             out_ref, xg, partial, snd, rs_buf, ctx_ref,
             ag_send, ag_recv, rs_send, rs_recv):
        my = lax.axis_index("i")
        left = lax.rem(my + N_DEV - 1, N_DEV)
        right = lax.rem(my + 1, N_DEV)

        barrier = pltpu.get_barrier_semaphore()
        for nbr in (left, right):
            pl.semaphore_signal(barrier, inc=1, device_id=(nbr,),
                                device_id_type=pl.DeviceIdType.MESH)
        pl.semaphore_wait(barrier, 2)

        cos_v = cos_ref[...]
        sin_v = sin_ref[...]
        wq = wq_ref[...].astype(jnp.bfloat16)
        wk = wk_ref[...].astype(jnp.bfloat16)
        wv = wv_ref[...].astype(jnp.bfloat16)
        wo = wo_ref[...].astype(jnp.bfloat16)

        xg[pl.ds(my, 1)] = x_ref[...].astype(jnp.bfloat16)[None]

        for h in range(N_DEV - 1):
            c_send = lax.rem(my + N_DEV - h, N_DEV)
            rdma = pltpu.make_async_remote_copy(
                src_ref=xg.at[c_send],
                dst_ref=xg.at[c_send],
                send_sem=ag_send.at[h],
                recv_sem=ag_recv.at[h],
                device_id=(right,),
                device_id_type=pl.DeviceIdType.MESH,
            )
            rdma.start()
            rdma.wait()

        col_ids = lax.broadcasted_iota(jnp.int32, (SQ, DM), 1)
        even = (col_ids % 2) == 0

        def rope(t):
            t_r = jnp.where(even, -pltpu.roll(t, -1, 1), pltpu.roll(t, 1, 1))
            return (t * cos_v + t_r * sin_v).astype(jnp.bfloat16)

        def compute_chunk(c, _):
            xb = xg[c]
            q = rope(jnp.dot(xb, wq, preferred_element_type=jnp.float32))
            k = rope(jnp.dot(xb, wk, preferred_element_type=jnp.float32))
            v = jnp.dot(xb, wv, preferred_element_type=jnp.float32
                        ).astype(jnp.bfloat16)

            def head(h, _):
                qh = q[:, pl.ds(h * DH, DH)]
                kh = k[:, pl.ds(h * DH, DH)]
                vh = v[:, pl.ds(h * DH, DH)]
                s = lax.dot_general(
                    qh, kh, (((1,), (1,)), ((), ())),
                    preferred_element_type=jnp.float32) * SCALE
                m = jnp.max(s, axis=-1, keepdims=True)
                w = jnp.exp(s - m)
                w = (w / jnp.sum(w, axis=-1, keepdims=True)).astype(jnp.bfloat16)
                ctx_ref[:, pl.ds(h * DH, DH)] = jnp.dot(
                    w, vh, preferred_element_type=jnp.float32
                ).astype(jnp.bfloat16)
                return 0

            lax.fori_loop(0, HL, head, 0)
            partial[pl.ds(c, 1)] = jnp.dot(
                ctx_ref[...], wo, preferred_element_type=jnp.float32)[None]
            return 0

        lax.fori_loop(0, N_DEV, compute_chunk, 0)

        for s in range(N_DEV - 1):
            c = lax.rem(my + N_DEV - 1 - s, N_DEV)
            if s == 0:
                snd[...] = partial[c].astype(jnp.bfloat16)
            else:
                snd[...] = (rs_buf[s - 1].astype(jnp.float32)
                            + partial[c]).astype(jnp.bfloat16)
            rdma = pltpu.make_async_remote_copy(
                src_ref=snd,
                dst_ref=rs_buf.at[s],
                send_sem=rs_send.at[s],
                recv_sem=rs_recv.at[s],
                device_id=(right,),
                device_id_type=pl.DeviceIdType.MESH,
            )
            rdma.start()
            rdma.wait()

        out_ref[...] = (rs_buf[N_DEV - 2].astype(jnp.float32)
                        + partial[my])[None]

    return pl.pallas_call(
        body,
        out_shape=jax.ShapeDtypeStruct((1, SQ, DM), jnp.float32),
        in_specs=[pl.BlockSpec(memory_space=pltpu.VMEM)] * 7,
        out_specs=pl.BlockSpec(memory_space=pltpu.VMEM),
        scratch_shapes=[
            pltpu.VMEM((N_DEV, SQ, DM), jnp.bfloat16),
            pltpu.VMEM((N_DEV, SQ, DM), jnp.float32),
            pltpu.VMEM((SQ, DM), jnp.bfloat16),
            pltpu.VMEM((N_DEV - 1, SQ, DM), jnp.bfloat16),
            pltpu.VMEM((SQ, DM), jnp.bfloat16),
            pltpu.SemaphoreType.DMA((N_DEV - 1,)),
            pltpu.SemaphoreType.DMA((N_DEV - 1,)),
            pltpu.SemaphoreType.DMA((N_DEV - 1,)),
            pltpu.SemaphoreType.DMA((N_DEV - 1,)),
        ],
        compiler_params=pltpu.CompilerParams(collective_id=0),
    )(x2, Wq, Wk, Wv, Wo, cos, sin)


# baseline (device time: 300578 ns/iter reference)
import numpy as np
import jax
import jax.numpy as jnp
from jax import lax
from jax.experimental import pallas as pl
from jax.experimental.pallas import tpu as pltpu

N_DEV = 4
B, SQ, DM = 4, 1024, 1024
HL, DH = 8, 128
SCALE = 0.08838834764831843


def _rope_tables():
    inv = 1.0 / (10000.0 ** (np.arange(0, DH, 2) / DH))
    pos = np.arange(SQ)[:, None] * inv[None, :]
    cos = np.repeat(np.cos(pos), 2, axis=-1).astype(np.float32)
    sin = np.repeat(np.sin(pos), 2, axis=-1).astype(np.float32)
    return np.tile(cos, (1, HL)), np.tile(sin, (1, HL))


_COS_NP, _SIN_NP = _rope_tables()


def kernel(x, Wq, Wk, Wv, Wo):
    x2 = x.reshape(SQ, DM).astype(jnp.bfloat16)
    wq = Wq.astype(jnp.bfloat16)
    wk = Wk.astype(jnp.bfloat16)
    wv = Wv.astype(jnp.bfloat16)
    wo = Wo.astype(jnp.bfloat16)
    cos = jnp.asarray(_COS_NP, dtype=jnp.bfloat16)
    sin = jnp.asarray(_SIN_NP, dtype=jnp.bfloat16)

    def body(x_ref, wq_ref, wk_ref, wv_ref, wo_ref, cos_ref, sin_ref,
             out_ref, xg, partial, ctx_ref,
             ag_send, ag_recv, rs_send, rs_recv):
        my = lax.axis_index("i")
        left = lax.rem(my + N_DEV - 1, N_DEV)
        right = lax.rem(my + 1, N_DEV)
        barrier = pltpu.get_barrier_semaphore()

        def ring_barrier():
            for nbr in (left, right):
                pl.semaphore_signal(barrier, inc=1, device_id=(nbr,),
                                    device_id_type=pl.DeviceIdType.MESH)
            pl.semaphore_wait(barrier, 2)

        ring_barrier()

        xg[pl.ds(my, 1)] = x_ref[...][None]

        for h in range(N_DEV - 1):
            c_send = lax.rem(my + N_DEV - h, N_DEV)
            rdma = pltpu.make_async_remote_copy(
                src_ref=xg.at[c_send],
                dst_ref=xg.at[c_send],
                send_sem=ag_send.at[h],
                recv_sem=ag_recv.at[h],
                device_id=(right,),
                device_id_type=pl.DeviceIdType.MESH,
            )
            rdma.start()
            rdma.wait()

        col_ids = lax.broadcasted_iota(jnp.int32, (SQ, DM), 1)
        even = (col_ids % 2) == 0

        def rope(t):
            t_r = jnp.where(even, -pltpu.roll(t, DM - 1, 1),
                            pltpu.roll(t, 1, 1))
            return (t * cos_ref[...].astype(jnp.float32)
                    + t_r * sin_ref[...].astype(jnp.float32)
                    ).astype(jnp.bfloat16)

        def compute_chunk(c, _):
            xb = xg[c]
            q = rope(jnp.dot(xb, wq_ref[...],
                             preferred_element_type=jnp.float32))
            k = rope(jnp.dot(xb, wk_ref[...],
                             preferred_element_type=jnp.float32))
            v = jnp.dot(xb, wv_ref[...],
                        preferred_element_type=jnp.float32).astype(jnp.bfloat16)

            for h in range(HL):
                qh = q[:, h * DH:(h + 1) * DH]
                kh = k[:, h * DH:(h + 1) * DH]
                vh = v[:, h * DH:(h + 1) * DH]
                s = lax.dot_general(
                    qh, kh, (((1,), (1,)), ((), ())),
                    preferred_element_type=jnp.float32) * SCALE
                m = jnp.max(s, axis=-1, keepdims=True)
                w = jnp.exp(s - m)
                w = (w / jnp.sum(w, axis=-1, keepdims=True)).astype(jnp.bfloat16)
                ctx_ref[:, h * DH:(h + 1) * DH] = jnp.dot(
                    w, vh, preferred_element_type=jnp.float32
                ).astype(jnp.bfloat16)

            partial[pl.ds(c, 1)] = jnp.dot(
                ctx_ref[...], wo_ref[...],
                preferred_element_type=jnp.float32).astype(jnp.bfloat16)[None]
            return 0

        lax.fori_loop(0, N_DEV, compute_chunk, 0)

        ring_barrier()

        for s in range(N_DEV - 1):
            c = lax.rem(my + N_DEV - 1 - s, N_DEV)
            if s == 0:
                xg[pl.ds(N_DEV - 1, 1)] = partial[pl.ds(c, 1)]
            else:
                xg[pl.ds(N_DEV - 1, 1)] = (
                    xg[s - 1].astype(jnp.float32)
                    + partial[c].astype(jnp.float32)
                ).astype(jnp.bfloat16)[None]
            rdma = pltpu.make_async_remote_copy(
                src_ref=xg.at[N_DEV - 1],
                dst_ref=xg.at[s],
                send_sem=rs_send.at[s],
                recv_sem=rs_recv.at[s],
                device_id=(right,),
                device_id_type=pl.DeviceIdType.MESH,
            )
            rdma.start()
            rdma.wait()

        out_ref[...] = (xg[N_DEV - 2].astype(jnp.float32)
                        + partial[my].astype(jnp.float32))[None]

    return pl.pallas_call(
        body,
        out_shape=jax.ShapeDtypeStruct((1, SQ, DM), jnp.float32),
        in_specs=[pl.BlockSpec(memory_space=pltpu.VMEM)] * 7,
        out_specs=pl.BlockSpec(memory_space=pltpu.VMEM),
        scratch_shapes=[
            pltpu.VMEM((N_DEV, SQ, DM), jnp.bfloat16),
            pltpu.VMEM((N_DEV, SQ, DM), jnp.bfloat16),
            pltpu.VMEM((SQ, DM), jnp.bfloat16),
            pltpu.SemaphoreType.DMA((N_DEV - 1,)),
            pltpu.SemaphoreType.DMA((N_DEV - 1,)),
            pltpu.SemaphoreType.DMA((N_DEV - 1,)),
            pltpu.SemaphoreType.DMA((N_DEV - 1,)),
        ],
        compiler_params=pltpu.CompilerParams(
            collective_id=0,
            vmem_limit_bytes=100 * 1024 * 1024,
        ),
    )(x2, wq, wk, wv, wo, cos, sin)


# device time: 196547 ns/iter; 1.5293x vs baseline; 1.5293x over previous
import numpy as np
import jax
import jax.numpy as jnp
from jax import lax
from jax.experimental import pallas as pl
from jax.experimental.pallas import tpu as pltpu

N_DEV = 4
B, SQ, DM = 4, 1024, 1024
HL, DH = 8, 128
SCALE = 0.08838834764831843


def _rope_tables():
    inv = 1.0 / (10000.0 ** (np.arange(0, DH, 2) / DH))
    pos = np.arange(SQ)[:, None] * inv[None, :]
    cos = np.repeat(np.cos(pos), 2, axis=-1).astype(np.float32)
    sin = np.repeat(np.sin(pos), 2, axis=-1).astype(np.float32)
    return np.tile(cos, (1, HL)), np.tile(sin, (1, HL))


_COS_NP, _SIN_NP = _rope_tables()


def kernel(x, Wq, Wk, Wv, Wo):
    x2 = x.reshape(SQ, DM).astype(jnp.bfloat16)
    wq = Wq.astype(jnp.bfloat16)
    wk = Wk.astype(jnp.bfloat16)
    wv = Wv.astype(jnp.bfloat16)
    wo = Wo.astype(jnp.bfloat16)
    cos = jnp.asarray(_COS_NP, dtype=jnp.bfloat16)
    sin = jnp.asarray(_SIN_NP, dtype=jnp.bfloat16)

    def body(x_ref, wq_ref, wk_ref, wv_ref, wo_ref, cos_ref, sin_ref,
             out_ref, xg, rs_rcv, rs_snd, ctx_ref,
             ag_send, ag_recv, rs_send, rs_recv):
        my = lax.axis_index("i")
        left = lax.rem(my + N_DEV - 1, N_DEV)
        right = lax.rem(my + 1, N_DEV)

        barrier = pltpu.get_barrier_semaphore()
        for nbr in (left, right):
            pl.semaphore_signal(barrier, inc=1, device_id=(nbr,),
                                device_id_type=pl.DeviceIdType.MESH)
        pl.semaphore_wait(barrier, 2)

        xg[pl.ds(my, 1)] = x_ref[...][None]

        col_ids = lax.broadcasted_iota(jnp.int32, (SQ, DM), 1)
        even = (col_ids % 2) == 0

        def rope(t):
            t_r = jnp.where(even, -pltpu.roll(t, DM - 1, 1),
                            pltpu.roll(t, 1, 1))
            return (t * cos_ref[...].astype(jnp.float32)
                    + t_r * sin_ref[...].astype(jnp.float32)
                    ).astype(jnp.bfloat16)

        def compute_chunk(c):
            xb = xg[c]
            q = rope(jnp.dot(xb, wq_ref[...],
                             preferred_element_type=jnp.float32))
            k = rope(jnp.dot(xb, wk_ref[...],
                             preferred_element_type=jnp.float32))
            v = jnp.dot(xb, wv_ref[...],
                        preferred_element_type=jnp.float32).astype(jnp.bfloat16)
            for h in range(HL):
                qh = q[:, h * DH:(h + 1) * DH]
                kh = k[:, h * DH:(h + 1) * DH]
                vh = v[:, h * DH:(h + 1) * DH]
                s = lax.dot_general(
                    qh, kh, (((1,), (1,)), ((), ())),
                    preferred_element_type=jnp.float32) * SCALE
                m = jnp.max(s, axis=-1, keepdims=True)
                w = jnp.exp(s - m)
                w = (w / jnp.sum(w, axis=-1, keepdims=True)).astype(jnp.bfloat16)
                ctx_ref[:, h * DH:(h + 1) * DH] = jnp.dot(
                    w, vh, preferred_element_type=jnp.float32
                ).astype(jnp.bfloat16)
            return jnp.dot(ctx_ref[...], wo_ref[...],
                           preferred_element_type=jnp.float32)

        def ag_hop(h):
            c_send = lax.rem(my + N_DEV - h, N_DEV)
            r = pltpu.make_async_remote_copy(
                src_ref=xg.at[c_send],
                dst_ref=xg.at[c_send],
                send_sem=ag_send.at[h],
                recv_sem=ag_recv.at[h],
                device_id=(right,),
                device_id_type=pl.DeviceIdType.MESH,
            )
            r.start()
            return r

        def rs_hop(s):
            r = pltpu.make_async_remote_copy(
                src_ref=rs_snd,
                dst_ref=rs_rcv.at[s],
                send_sem=rs_send.at[s],
                recv_sem=rs_recv.at[s],
                device_id=(right,),
                device_id_type=pl.DeviceIdType.MESH,
            )
            r.start()
            return r

        a0 = ag_hop(0)
        out_ref[...] = compute_chunk(my)[None]

        a0.wait_recv()
        a1 = ag_hop(1)
        p = compute_chunk(lax.rem(my + 3, N_DEV))
        rs_snd[...] = p.astype(jnp.bfloat16)
        r0 = rs_hop(0)

        a1.wait_recv()
        a2 = ag_hop(2)
        p = compute_chunk(lax.rem(my + 2, N_DEV))
        r0.wait_recv()
        r0.wait_send()
        rs_snd[...] = (rs_rcv[0].astype(jnp.float32) + p).astype(jnp.bfloat16)
        r1 = rs_hop(1)

        a2.wait_recv()
        p = compute_chunk(lax.rem(my + 1, N_DEV))
        r1.wait_recv()
        r1.wait_send()
        rs_snd[...] = (rs_rcv[1].astype(jnp.float32) + p).astype(jnp.bfloat16)
        r2 = rs_hop(2)

        r2.wait_recv()
        out_ref[...] = out_ref[...] + rs_rcv[pl.ds(2, 1)].astype(jnp.float32)

        a0.wait_send()
        a1.wait_send()
        a2.wait_send()
        r2.wait_send()

    return pl.pallas_call(
        body,
        out_shape=jax.ShapeDtypeStruct((1, SQ, DM), jnp.float32),
        in_specs=[pl.BlockSpec(memory_space=pltpu.VMEM)] * 7,
        out_specs=pl.BlockSpec(memory_space=pltpu.VMEM),
        scratch_shapes=[
            pltpu.VMEM((N_DEV, SQ, DM), jnp.bfloat16),
            pltpu.VMEM((N_DEV - 1, SQ, DM), jnp.bfloat16),
            pltpu.VMEM((SQ, DM), jnp.bfloat16),
            pltpu.VMEM((SQ, DM), jnp.bfloat16),
            pltpu.SemaphoreType.DMA((N_DEV - 1,)),
            pltpu.SemaphoreType.DMA((N_DEV - 1,)),
            pltpu.SemaphoreType.DMA((N_DEV - 1,)),
            pltpu.SemaphoreType.DMA((N_DEV - 1,)),
        ],
        compiler_params=pltpu.CompilerParams(
            collective_id=0,
            vmem_limit_bytes=100 * 1024 * 1024,
        ),
    )(x2, wq, wk, wv, wo, cos, sin)


# device time: 170359 ns/iter; 1.7644x vs baseline; 1.1537x over previous
import numpy as np
import jax
import jax.numpy as jnp
from jax import lax
from jax.experimental import pallas as pl
from jax.experimental.pallas import tpu as pltpu

N_DEV = 4
B, SQ, DM = 4, 1024, 1024
HL, DH = 8, 128
SCALE = 0.08838834764831843


def _rope_tables():
    inv = 1.0 / (10000.0 ** (np.arange(0, DH, 2) / DH))
    pos = np.arange(SQ)[:, None] * inv[None, :]
    cos = np.repeat(np.cos(pos), 2, axis=-1).astype(np.float32)
    sin = np.repeat(np.sin(pos), 2, axis=-1).astype(np.float32)
    return np.tile(cos, (1, HL)), np.tile(sin, (1, HL))


_COS_NP, _SIN_NP = _rope_tables()


def kernel(x, Wq, Wk, Wv, Wo):
    x2 = x.reshape(SQ, DM).astype(jnp.bfloat16)
    wq = (Wq * SCALE).astype(jnp.bfloat16)
    wk = Wk.astype(jnp.bfloat16)
    wv = Wv.astype(jnp.bfloat16)
    wo = Wo.astype(jnp.bfloat16)
    cos = jnp.asarray(_COS_NP, dtype=jnp.bfloat16)
    sin = jnp.asarray(_SIN_NP, dtype=jnp.bfloat16)

    def body(x_ref, wq_ref, wk_ref, wv_ref, wo_ref, cos_ref, sin_ref,
             out_ref, xg, rs_rcv, rs_snd, ctx_ref,
             ag_send, ag_recv, rs_send, rs_recv):
        my = lax.axis_index("i")
        left = lax.rem(my + N_DEV - 1, N_DEV)
        right = lax.rem(my + 1, N_DEV)

        barrier = pltpu.get_barrier_semaphore()
        for nbr in (left, right):
            pl.semaphore_signal(barrier, inc=1, device_id=(nbr,),
                                device_id_type=pl.DeviceIdType.MESH)
        pl.semaphore_wait(barrier, 2)

        xg[pl.ds(my, 1)] = x_ref[...][None]

        col_ids = lax.broadcasted_iota(jnp.int32, (SQ, DM), 1)
        even = (col_ids % 2) == 0

        def rope(t):
            t_r = jnp.where(even, -pltpu.roll(t, DM - 1, 1),
                            pltpu.roll(t, 1, 1))
            return (t * cos_ref[...].astype(jnp.float32)
                    + t_r * sin_ref[...].astype(jnp.float32)
                    ).astype(jnp.bfloat16)

        def compute_chunk(c):
            xb = xg[c]
            q = rope(jnp.dot(xb, wq_ref[...],
                             preferred_element_type=jnp.float32))
            k = rope(jnp.dot(xb, wk_ref[...],
                             preferred_element_type=jnp.float32))
            v = jnp.dot(xb, wv_ref[...],
                        preferred_element_type=jnp.float32).astype(jnp.bfloat16)
            for h in range(HL):
                qh = q[:, h * DH:(h + 1) * DH]
                kh = k[:, h * DH:(h + 1) * DH]
                vh = v[:, h * DH:(h + 1) * DH]
                s = lax.dot_general(
                    qh, kh, (((1,), (1,)), ((), ())),
                    preferred_element_type=jnp.float32)
                w = jnp.exp(s)
                denom = jnp.sum(w, axis=-1, keepdims=True)
                ctx_ref[:, h * DH:(h + 1) * DH] = (
                    jnp.dot(w.astype(jnp.bfloat16), vh,
                            preferred_element_type=jnp.float32) / denom
                ).astype(jnp.bfloat16)
            return jnp.dot(ctx_ref[...], wo_ref[...],
                           preferred_element_type=jnp.float32)

        def ag_hop(h):
            c_send = lax.rem(my + N_DEV - h, N_DEV)
            r = pltpu.make_async_remote_copy(
                src_ref=xg.at[c_send],
                dst_ref=xg.at[c_send],
                send_sem=ag_send.at[h],
                recv_sem=ag_recv.at[h],
                device_id=(right,),
                device_id_type=pl.DeviceIdType.MESH,
            )
            r.start()
            return r

        def rs_hop(s):
            r = pltpu.make_async_remote_copy(
                src_ref=rs_snd,
                dst_ref=rs_rcv.at[s],
                send_sem=rs_send.at[s],
                recv_sem=rs_recv.at[s],
                device_id=(right,),
                device_id_type=pl.DeviceIdType.MESH,
            )
            r.start()
            return r

        a0 = ag_hop(0)
        out_ref[...] = compute_chunk(my)[None]

        a0.wait_recv()
        a1 = ag_hop(1)
        p = compute_chunk(lax.rem(my + 3, N_DEV))
        rs_snd[...] = p.astype(jnp.bfloat16)
        r0 = rs_hop(0)

        a1.wait_recv()
        a2 = ag_hop(2)
        p = compute_chunk(lax.rem(my + 2, N_DEV))
        r0.wait_recv()
        r0.wait_send()
        rs_snd[...] = (rs_rcv[0].astype(jnp.float32) + p).astype(jnp.bfloat16)
        r1 = rs_hop(1)

        a2.wait_recv()
        p = compute_chunk(lax.rem(my + 1, N_DEV))
        r1.wait_recv()
        r1.wait_send()
        rs_snd[...] = (rs_rcv[1].astype(jnp.float32) + p).astype(jnp.bfloat16)
        r2 = rs_hop(2)

        r2.wait_recv()
        out_ref[...] = out_ref[...] + rs_rcv[pl.ds(2, 1)].astype(jnp.float32)

        a0.wait_send()
        a1.wait_send()
        a2.wait_send()
        r2.wait_send()

    return pl.pallas_call(
        body,
        out_shape=jax.ShapeDtypeStruct((1, SQ, DM), jnp.float32),
        in_specs=[pl.BlockSpec(memory_space=pltpu.VMEM)] * 7,
        out_specs=pl.BlockSpec(memory_space=pltpu.VMEM),
        scratch_shapes=[
            pltpu.VMEM((N_DEV, SQ, DM), jnp.bfloat16),
            pltpu.VMEM((N_DEV - 1, SQ, DM), jnp.bfloat16),
            pltpu.VMEM((SQ, DM), jnp.bfloat16),
            pltpu.VMEM((SQ, DM), jnp.bfloat16),
            pltpu.SemaphoreType.DMA((N_DEV - 1,)),
            pltpu.SemaphoreType.DMA((N_DEV - 1,)),
            pltpu.SemaphoreType.DMA((N_DEV - 1,)),
            pltpu.SemaphoreType.DMA((N_DEV - 1,)),
        ],
        compiler_params=pltpu.CompilerParams(
            collective_id=0,
            vmem_limit_bytes=100 * 1024 * 1024,
        ),
    )(x2, wq, wk, wv, wo, cos, sin)


# device time: 137248 ns/iter; 2.1900x vs baseline; 1.2412x over previous
import numpy as np
import jax
import jax.numpy as jnp
from jax import lax
from jax.experimental import pallas as pl
from jax.experimental.pallas import tpu as pltpu

N_DEV = 4
B, SQ, DM = 4, 1024, 1024
SQH = SQ // 2
HL, DH = 8, 128
SCALE = 0.08838834764831843


def _rope_tables():
    inv = 1.0 / (10000.0 ** (np.arange(0, DH, 2) / DH))
    pos = np.arange(SQ)[:, None] * inv[None, :]
    cos = np.repeat(np.cos(pos), 2, axis=-1).astype(np.float32)
    sin = np.repeat(np.sin(pos), 2, axis=-1).astype(np.float32)
    return np.tile(cos, (1, HL)), np.tile(sin, (1, HL))


_COS_NP, _SIN_NP = _rope_tables()


def kernel(x, Wq, Wk, Wv, Wo):
    x2 = x.reshape(SQ, DM).astype(jnp.bfloat16)
    wq = (Wq * SCALE).astype(jnp.bfloat16)
    wk = Wk.astype(jnp.bfloat16)
    wv = Wv.astype(jnp.bfloat16)
    wo = Wo.astype(jnp.bfloat16)
    cos = jnp.asarray(_COS_NP, dtype=jnp.bfloat16)
    sin = jnp.asarray(_SIN_NP, dtype=jnp.bfloat16)

    def body(x_ref, wq_ref, wk_ref, wv_ref, wo_ref, cos_ref, sin_ref,
             out_ref, xg, rcv_t, rcv_b, snd_t, snd_b, pp1_top, pm1_bot,
             ctx_ref, ag_send, ag_recv, rst_send, rst_recv,
             rsb_send, rsb_recv):
        my = lax.axis_index("i")
        left = lax.rem(my + N_DEV - 1, N_DEV)
        right = lax.rem(my + 1, N_DEV)

        barrier = pltpu.get_barrier_semaphore()
        for nbr in (left, right):
            pl.semaphore_signal(barrier, inc=1, device_id=(nbr,),
                                device_id_type=pl.DeviceIdType.MESH)
        pl.semaphore_wait(barrier, 2)

        xg[pl.ds(my, 1)] = x_ref[...][None]

        col_ids = lax.broadcasted_iota(jnp.int32, (SQ, DM), 1)
        even = (col_ids % 2) == 0

        def rope(t):
            t_r = jnp.where(even, -pltpu.roll(t, DM - 1, 1),
                            pltpu.roll(t, 1, 1))
            return t * cos_ref[...] + t_r * sin_ref[...]

        def compute_chunk(c):
            xb = xg[c]
            q = rope(jnp.dot(xb, wq_ref[...],
                             preferred_element_type=jnp.float32
                             ).astype(jnp.bfloat16))
            k = rope(jnp.dot(xb, wk_ref[...],
                             preferred_element_type=jnp.float32
                             ).astype(jnp.bfloat16))
            v = jnp.dot(xb, wv_ref[...],
                        preferred_element_type=jnp.float32
                        ).astype(jnp.bfloat16)
            for h in range(HL):
                qh = q[:, h * DH:(h + 1) * DH]
                kh = k[:, h * DH:(h + 1) * DH]
                vh = v[:, h * DH:(h + 1) * DH]
                s = lax.dot_general(
                    qh, kh, (((1,), (1,)), ((), ())),
                    preferred_element_type=jnp.float32)
                w = jnp.exp(s).astype(jnp.bfloat16)
                denom = jnp.sum(w.astype(jnp.float32), axis=-1, keepdims=True)
                ctx_ref[:, h * DH:(h + 1) * DH] = (
                    jnp.dot(w, vh, preferred_element_type=jnp.float32) / denom
                ).astype(jnp.bfloat16)
            return jnp.dot(ctx_ref[...], wo_ref[...],
                           preferred_element_type=jnp.float32)

        def ag_rdma(slot, dev, h):
            r = pltpu.make_async_remote_copy(
                src_ref=xg.at[slot],
                dst_ref=xg.at[slot],
                send_sem=ag_send.at[h],
                recv_sem=ag_recv.at[h],
                device_id=(dev,),
                device_id_type=pl.DeviceIdType.MESH,
            )
            r.start()
            return r

        def rs_rdma(src, dst_arr, s, dev, send_sems, recv_sems):
            r = pltpu.make_async_remote_copy(
                src_ref=src,
                dst_ref=dst_arr.at[s],
                send_sem=send_sems.at[s],
                recv_sem=recv_sems.at[s],
                device_id=(dev,),
                device_id_type=pl.DeviceIdType.MESH,
            )
            r.start()
            return r

        aL = ag_rdma(my, left, 2)
        a0 = ag_rdma(my, right, 0)

        out_ref[...] = compute_chunk(my)[None]

        a0.wait_recv()
        a1 = ag_rdma(lax.rem(my + 3, N_DEV), right, 1)

        p = compute_chunk(lax.rem(my + 3, N_DEV))
        snd_t[...] = p[:SQH].astype(jnp.bfloat16)
        pm1_bot[...] = p[SQH:].astype(jnp.bfloat16)
        rt0 = rs_rdma(snd_t, rcv_t, 0, right, rst_send, rst_recv)

        aL.wait_recv()
        p = compute_chunk(lax.rem(my + 1, N_DEV))
        snd_b[...] = p[SQH:].astype(jnp.bfloat16)
        pp1_top[...] = p[:SQH].astype(jnp.bfloat16)
        rb0 = rs_rdma(snd_b, rcv_b, 0, left, rsb_send, rsb_recv)

        a1.wait_recv()
        p = compute_chunk(lax.rem(my + 2, N_DEV))

        rt0.wait_recv()
        rt0.wait_send()
        snd_t[...] = (rcv_t[0].astype(jnp.float32)
                      + p[:SQH]).astype(jnp.bfloat16)
        rt1 = rs_rdma(snd_t, rcv_t, 1, right, rst_send, rst_recv)

        rb0.wait_recv()
        rb0.wait_send()
        snd_b[...] = (rcv_b[0].astype(jnp.float32)
                      + p[SQH:]).astype(jnp.bfloat16)
        rb1 = rs_rdma(snd_b, rcv_b, 1, left, rsb_send, rsb_recv)

        rt1.wait_recv()
        rt1.wait_send()
        snd_t[...] = (rcv_t[1].astype(jnp.float32)
                      + pp1_top[...].astype(jnp.float32)).astype(jnp.bfloat16)
        rt2 = rs_rdma(snd_t, rcv_t, 2, right, rst_send, rst_recv)

        rb1.wait_recv()
        rb1.wait_send()
        snd_b[...] = (rcv_b[1].astype(jnp.float32)
                      + pm1_bot[...].astype(jnp.float32)).astype(jnp.bfloat16)
        rb2 = rs_rdma(snd_b, rcv_b, 2, left, rsb_send, rsb_recv)

        rt2.wait_recv()
        out_ref[0, :SQH, :] = (out_ref[0, :SQH, :]
                               + rcv_t[2].astype(jnp.float32))
        rb2.wait_recv()
        out_ref[0, SQH:, :] = (out_ref[0, SQH:, :]
                               + rcv_b[2].astype(jnp.float32))

        aL.wait_send()
        a0.wait_send()
        a1.wait_send()
        rt2.wait_send()
        rb2.wait_send()

    return pl.pallas_call(
        body,
        out_shape=jax.ShapeDtypeStruct((1, SQ, DM), jnp.float32),
        in_specs=[pl.BlockSpec(memory_space=pltpu.VMEM)] * 7,
        out_specs=pl.BlockSpec(memory_space=pltpu.VMEM),
        scratch_shapes=[
            pltpu.VMEM((N_DEV, SQ, DM), jnp.bfloat16),
            pltpu.VMEM((N_DEV - 1, SQH, DM), jnp.bfloat16),
            pltpu.VMEM((N_DEV - 1, SQH, DM), jnp.bfloat16),
            pltpu.VMEM((SQH, DM), jnp.bfloat16),
            pltpu.VMEM((SQH, DM), jnp.bfloat16),
            pltpu.VMEM((SQH, DM), jnp.bfloat16),
            pltpu.VMEM((SQH, DM), jnp.bfloat16),
            pltpu.VMEM((SQ, DM), jnp.bfloat16),
            pltpu.SemaphoreType.DMA((3,)),
            pltpu.SemaphoreType.DMA((3,)),
            pltpu.SemaphoreType.DMA((3,)),
            pltpu.SemaphoreType.DMA((3,)),
            pltpu.SemaphoreType.DMA((3,)),
            pltpu.SemaphoreType.DMA((3,)),
        ],
        compiler_params=pltpu.CompilerParams(
            collective_id=0,
            vmem_limit_bytes=100 * 1024 * 1024,
        ),
    )(x2, wq, wk, wv, wo, cos, sin)
